# baseline (speedup 1.0000x reference)
import functools

import jax
import jax.numpy as jnp
from jax.experimental import pallas as pl
from jax.experimental.pallas import tpu as pltpu

_BN = 1024
_BE = 1600


def _rup(x, m):
    return (x + m - 1) // m * m


def _matmul_bias_body(x_ref, wt_ref, b_ref, o_ref):
    o_ref[...] = (jnp.dot(x_ref[...], wt_ref[...],
                          preferred_element_type=jnp.float32) + b_ref[...])


def _tc_matmul_bias(x, wt, b):
    n, d1 = x.shape
    d2 = wt.shape[1]
    bn = min(_BN, n)
    return pl.pallas_call(
        _matmul_bias_body,
        grid=(n // bn,),
        in_specs=[pl.BlockSpec((bn, d1), lambda i: (i, 0)),
                  pl.BlockSpec((d1, d2), lambda i: (0, 0)),
                  pl.BlockSpec((1, d2), lambda i: (0, 0))],
        out_specs=pl.BlockSpec((bn, d2), lambda i: (i, 0)),
        out_shape=jax.ShapeDtypeStruct((n, d2), jnp.float32),
    )(x, wt, b.reshape(1, -1))


def _edge_mlp_body_pre(pre_ref, ea_ref, w1et_ref, b1_ref, w2t_ref, b2_ref,
                       w3t_ref, b3_ref, w4t_ref, b4_ref, o_ref):
    x = (jnp.dot(ea_ref[...], w1et_ref[...], preferred_element_type=jnp.float32)
         + b1_ref[...] + pre_ref[...])
    hh = jax.nn.relu(x)
    hh = jax.nn.relu(jnp.dot(hh, w2t_ref[...], preferred_element_type=jnp.float32)
                     + b2_ref[...])
    hh = jax.nn.relu(jnp.dot(hh, w3t_ref[...], preferred_element_type=jnp.float32)
                     + b3_ref[...])
    o_ref[...] = (jnp.dot(hh, w4t_ref[...], preferred_element_type=jnp.float32)
                  + b4_ref[...])


def _edge_mlp_body_nopre(ea_ref, w1et_ref, b1_ref, w2t_ref, b2_ref,
                         w3t_ref, b3_ref, w4t_ref, b4_ref, o_ref):
    x = (jnp.dot(ea_ref[...], w1et_ref[...], preferred_element_type=jnp.float32)
         + b1_ref[...])
    hh = jax.nn.relu(x)
    hh = jax.nn.relu(jnp.dot(hh, w2t_ref[...], preferred_element_type=jnp.float32)
                     + b2_ref[...])
    hh = jax.nn.relu(jnp.dot(hh, w3t_ref[...], preferred_element_type=jnp.float32)
                     + b3_ref[...])
    o_ref[...] = (jnp.dot(hh, w4t_ref[...], preferred_element_type=jnp.float32)
                  + b4_ref[...])


def _tc_edge_mlp(h1pre, ea, w1et, b1, w2t, b2, w3t, b3, w4t, b4):
    e, ef = ea.shape
    msg = w1et.shape[1]
    be = min(_BE, e)
    grid = (e // be,)
    wspecs = [
        pl.BlockSpec((ef, msg), lambda i: (0, 0)),
        pl.BlockSpec((1, msg), lambda i: (0, 0)),
        pl.BlockSpec((msg, msg), lambda i: (0, 0)),
        pl.BlockSpec((1, msg), lambda i: (0, 0)),
        pl.BlockSpec((msg, msg), lambda i: (0, 0)),
        pl.BlockSpec((1, msg), lambda i: (0, 0)),
        pl.BlockSpec((msg, ef), lambda i: (0, 0)),
        pl.BlockSpec((1, ef), lambda i: (0, 0)),
    ]
    wargs = (w1et, b1.reshape(1, -1), w2t, b2.reshape(1, -1),
             w3t, b3.reshape(1, -1), w4t, b4.reshape(1, -1))
    out_spec = pl.BlockSpec((be, ef), lambda i: (i, 0))
    out_shape = jax.ShapeDtypeStruct((e, ef), jnp.float32)
    if h1pre is None:
        return pl.pallas_call(
            _edge_mlp_body_nopre,
            grid=grid,
            in_specs=[pl.BlockSpec((be, ef), lambda i: (i, 0))] + wspecs,
            out_specs=out_spec, out_shape=out_shape,
        )(ea, *wargs)
    hdim = h1pre.shape[1]
    return pl.pallas_call(
        _edge_mlp_body_pre,
        grid=grid,
        in_specs=[pl.BlockSpec((be, hdim), lambda i: (i, 0)),
                  pl.BlockSpec((be, ef), lambda i: (i, 0))] + wspecs,
        out_specs=out_spec, out_shape=out_shape,
    )(h1pre, ea, *wargs)


def _gru_core(agg_ref, gx0_ref, st_ref, wat_ref, whht_ref, bhh_ref):
    h = st_ref.shape[1]
    agg = jnp.sum(agg_ref[...], axis=0)
    gx = (jnp.dot(agg, wat_ref[...], preferred_element_type=jnp.float32)
          + gx0_ref[...])
    gh = (jnp.dot(st_ref[...], whht_ref[...], preferred_element_type=jnp.float32)
          + bhh_ref[...])
    r = jax.nn.sigmoid(gx[:, :h] + gh[:, :h])
    z = jax.nn.sigmoid(gx[:, h:2 * h] + gh[:, h:2 * h])
    nn_ = jnp.tanh(gx[:, 2 * h:] + r * gh[:, 2 * h:])
    return (1.0 - z) * nn_ + z * st_ref[...]


def _gru_proj_body(agg_ref, gx0_ref, st_ref, wat_ref, whht_ref, bhh_ref,
                   w1st_ref, w1dt_ref, ns_ref, ps_ref, pd_ref):
    ns = _gru_core(agg_ref, gx0_ref, st_ref, wat_ref, whht_ref, bhh_ref)
    ns_ref[...] = ns
    ps_ref[...] = jnp.dot(ns, w1st_ref[...], preferred_element_type=jnp.float32)
    pd_ref[...] = jnp.dot(ns, w1dt_ref[...], preferred_element_type=jnp.float32)


def _gru_final_body(agg_ref, gx0_ref, st_ref, wat_ref, whht_ref, bhh_ref,
                    wft_ref, bf_ref, lg_ref):
    ns = _gru_core(agg_ref, gx0_ref, st_ref, wat_ref, whht_ref, bhh_ref)
    lg_ref[...] = (jnp.dot(ns, wft_ref[...], preferred_element_type=jnp.float32)
                   + bf_ref[...])


def _tc_gru(agg, gx0, states, wat, whht, bhh, final, w1st=None, w1dt=None,
            wft=None, bf=None):
    p, n, ef = agg.shape
    h = states.shape[1]
    bn = min(_BN, n)
    grid = (n // bn,)
    base_specs = [
        pl.BlockSpec((p, bn, ef), lambda i: (0, i, 0)),
        pl.BlockSpec((bn, 3 * h), lambda i: (i, 0)),
        pl.BlockSpec((bn, h), lambda i: (i, 0)),
        pl.BlockSpec((ef, 3 * h), lambda i: (0, 0)),
        pl.BlockSpec((h, 3 * h), lambda i: (0, 0)),
        pl.BlockSpec((1, 3 * h), lambda i: (0, 0)),
    ]
    if not final:
        msg = w1st.shape[1]
        return pl.pallas_call(
            _gru_proj_body,
            grid=grid,
            in_specs=base_specs + [pl.BlockSpec((h, msg), lambda i: (0, 0)),
                                   pl.BlockSpec((h, msg), lambda i: (0, 0))],
            out_specs=[pl.BlockSpec((bn, h), lambda i: (i, 0)),
                       pl.BlockSpec((bn, msg), lambda i: (i, 0)),
                       pl.BlockSpec((bn, msg), lambda i: (i, 0))],
            out_shape=[jax.ShapeDtypeStruct((n, h), jnp.float32),
                       jax.ShapeDtypeStruct((n, msg), jnp.float32),
                       jax.ShapeDtypeStruct((n, msg), jnp.float32)],
        )(agg, gx0, states, wat, whht, bhh.reshape(1, -1), w1st, w1dt)
    nl = wft.shape[1]
    return pl.pallas_call(
        _gru_final_body,
        grid=grid,
        in_specs=base_specs + [pl.BlockSpec((h, nl), lambda i: (0, 0)),
                               pl.BlockSpec((1, nl), lambda i: (0, 0))],
        out_specs=pl.BlockSpec((bn, nl), lambda i: (i, 0)),
        out_shape=jax.ShapeDtypeStruct((n, nl), jnp.float32),
    )(agg, gx0, states, wat, whht, bhh.reshape(1, -1), wft, bf.reshape(1, -1))


def _gather_add(ps, pd, src, dst):
    return jnp.take(ps, src, axis=0) + jnp.take(pd, dst, axis=0)


def _scatter_add(msgs, dst, n):
    ef = msgs.shape[1]
    agg = jnp.zeros((n, ef), jnp.float32).at[dst].add(msgs)
    return agg[None]


def kernel(x_nodes, edge_index, edge_attr, node_mask, edge_mask,
           W1, b1, W2, b2, W3, b3, W4, b4, W_ih, b_ih, W_hh, b_hh, Wf, bf):
    n, h = x_nodes.shape
    e, ef = edge_attr.shape
    t_iters = 7
    src = edge_index[0].astype(jnp.int32)
    dst = edge_index[1].astype(jnp.int32)
    np_ = _rup(n, _BN)
    xp = jnp.pad(x_nodes, ((0, np_ - n), (0, 0)))

    w1st = W1[:, :h].T
    w1dt = W1[:, h:2 * h].T
    w1et = W1[:, 2 * h:].T
    wat = W_ih[:, :ef].T
    wxt = W_ih[:, ef:].T
    whht = W_hh.T
    wft = Wf.T

    gx0 = _tc_matmul_bias(xp, wxt, b_ih)

    states = jnp.zeros((np_, h), jnp.float32)
    ps = pd = None
    out = None
    for t in range(t_iters):
        if t == 0:
            msgs = _tc_edge_mlp(None, edge_attr, w1et, b1, W2.T, b2, W3.T, b3,
                                W4.T, b4)
        else:
            h1pre = _gather_add(ps, pd, src, dst)
            msgs = _tc_edge_mlp(h1pre, edge_attr, w1et, b1, W2.T, b2, W3.T, b3,
                                W4.T, b4)
        agg = _scatter_add(msgs, dst, np_)
        if t < t_iters - 1:
            states, ps, pd = _tc_gru(agg, gx0, states, wat, whht, b_hh,
                                     final=False, w1st=w1st, w1dt=w1dt)
        else:
            out = _tc_gru(agg, gx0, states, wat, whht, b_hh,
                          final=True, wft=wft, bf=bf)
    return out[:n]


# speedup vs baseline: 1.6093x; 1.6093x over previous
import functools

import jax
import jax.numpy as jnp
from jax import lax
from jax.experimental import pallas as pl
from jax.experimental.pallas import tpu as pltpu
from jax.experimental.pallas import tpu_sc as plsc

_BN = 1024
_BE = 1600
_NW = 32
_NT = 16
_CH = 128


def _rup(x, m):
    return (x + m - 1) // m * m


def _matmul_bias_body(x_ref, wt_ref, b_ref, o_ref):
    o_ref[...] = (jnp.dot(x_ref[...], wt_ref[...],
                          preferred_element_type=jnp.float32) + b_ref[...])


def _tc_matmul_bias(x, wt, b):
    n, d1 = x.shape
    d2 = wt.shape[1]
    bn = min(_BN, n)
    return pl.pallas_call(
        _matmul_bias_body,
        grid=(n // bn,),
        in_specs=[pl.BlockSpec((bn, d1), lambda i: (i, 0)),
                  pl.BlockSpec((d1, d2), lambda i: (0, 0)),
                  pl.BlockSpec((1, d2), lambda i: (0, 0))],
        out_specs=pl.BlockSpec((bn, d2), lambda i: (i, 0)),
        out_shape=jax.ShapeDtypeStruct((n, d2), jnp.float32),
    )(x, wt, b.reshape(1, -1))


def _edge_mlp_body_pre(pres_ref, pred_ref, ea_ref, w1et_ref, b1_ref, w2t_ref,
                       b2_ref, w3t_ref, b3_ref, w4t_ref, b4_ref, o_ref):
    x = (jnp.dot(ea_ref[...], w1et_ref[...], preferred_element_type=jnp.float32)
         + b1_ref[...] + pres_ref[...] + pred_ref[...])
    hh = jax.nn.relu(x)
    hh = jax.nn.relu(jnp.dot(hh, w2t_ref[...], preferred_element_type=jnp.float32)
                     + b2_ref[...])
    hh = jax.nn.relu(jnp.dot(hh, w3t_ref[...], preferred_element_type=jnp.float32)
                     + b3_ref[...])
    o_ref[...] = (jnp.dot(hh, w4t_ref[...], preferred_element_type=jnp.float32)
                  + b4_ref[...])


def _edge_mlp_body_nopre(ea_ref, w1et_ref, b1_ref, w2t_ref, b2_ref,
                         w3t_ref, b3_ref, w4t_ref, b4_ref, o_ref):
    x = (jnp.dot(ea_ref[...], w1et_ref[...], preferred_element_type=jnp.float32)
         + b1_ref[...])
    hh = jax.nn.relu(x)
    hh = jax.nn.relu(jnp.dot(hh, w2t_ref[...], preferred_element_type=jnp.float32)
                     + b2_ref[...])
    hh = jax.nn.relu(jnp.dot(hh, w3t_ref[...], preferred_element_type=jnp.float32)
                     + b3_ref[...])
    o_ref[...] = (jnp.dot(hh, w4t_ref[...], preferred_element_type=jnp.float32)
                  + b4_ref[...])


def _tc_edge_mlp(pres, pred, ea, w1et, b1, w2t, b2, w3t, b3, w4t, b4):
    e, ef = ea.shape
    msg = w1et.shape[1]
    be = min(_BE, e)
    grid = (e // be,)
    wspecs = [
        pl.BlockSpec((ef, msg), lambda i: (0, 0)),
        pl.BlockSpec((1, msg), lambda i: (0, 0)),
        pl.BlockSpec((msg, msg), lambda i: (0, 0)),
        pl.BlockSpec((1, msg), lambda i: (0, 0)),
        pl.BlockSpec((msg, msg), lambda i: (0, 0)),
        pl.BlockSpec((1, msg), lambda i: (0, 0)),
        pl.BlockSpec((msg, ef), lambda i: (0, 0)),
        pl.BlockSpec((1, ef), lambda i: (0, 0)),
    ]
    wargs = (w1et, b1.reshape(1, -1), w2t, b2.reshape(1, -1),
             w3t, b3.reshape(1, -1), w4t, b4.reshape(1, -1))
    out_spec = pl.BlockSpec((be, ef), lambda i: (i, 0))
    out_shape = jax.ShapeDtypeStruct((e, ef), jnp.float32)
    if pres is None:
        return pl.pallas_call(
            _edge_mlp_body_nopre,
            grid=grid,
            in_specs=[pl.BlockSpec((be, ef), lambda i: (i, 0))] + wspecs,
            out_specs=out_spec, out_shape=out_shape,
        )(ea, *wargs)
    hdim = pres.shape[1]
    return pl.pallas_call(
        _edge_mlp_body_pre,
        grid=grid,
        in_specs=[pl.BlockSpec((be, hdim), lambda i: (i, 0)),
                  pl.BlockSpec((be, hdim), lambda i: (i, 0)),
                  pl.BlockSpec((be, ef), lambda i: (i, 0))] + wspecs,
        out_specs=out_spec, out_shape=out_shape,
    )(pres, pred, ea, *wargs)


def _gru_core(agg_ref, gx0_ref, st_ref, wat_ref, whht_ref, bhh_ref):
    h = st_ref.shape[1]
    agg = jnp.sum(agg_ref[...], axis=0)
    gx = (jnp.dot(agg, wat_ref[...], preferred_element_type=jnp.float32)
          + gx0_ref[...])
    gh = (jnp.dot(st_ref[...], whht_ref[...], preferred_element_type=jnp.float32)
          + bhh_ref[...])
    r = jax.nn.sigmoid(gx[:, :h] + gh[:, :h])
    z = jax.nn.sigmoid(gx[:, h:2 * h] + gh[:, h:2 * h])
    nn_ = jnp.tanh(gx[:, 2 * h:] + r * gh[:, 2 * h:])
    return (1.0 - z) * nn_ + z * st_ref[...]


def _gru_proj_body(agg_ref, gx0_ref, st_ref, wat_ref, whht_ref, bhh_ref,
                   w1st_ref, w1dt_ref, ns_ref, ps_ref, pd_ref):
    ns = _gru_core(agg_ref, gx0_ref, st_ref, wat_ref, whht_ref, bhh_ref)
    ns_ref[...] = ns
    ps_ref[...] = jnp.dot(ns, w1st_ref[...], preferred_element_type=jnp.float32)
    pd_ref[...] = jnp.dot(ns, w1dt_ref[...], preferred_element_type=jnp.float32)


def _gru_final_body(agg_ref, gx0_ref, st_ref, wat_ref, whht_ref, bhh_ref,
                    wft_ref, bf_ref, lg_ref):
    ns = _gru_core(agg_ref, gx0_ref, st_ref, wat_ref, whht_ref, bhh_ref)
    lg_ref[...] = (jnp.dot(ns, wft_ref[...], preferred_element_type=jnp.float32)
                   + bf_ref[...])


def _tc_gru(agg, gx0, states, wat, whht, bhh, final, w1st=None, w1dt=None,
            wft=None, bf=None):
    p, n, ef = agg.shape
    h = states.shape[1]
    bn = min(_BN, n)
    grid = (n // bn,)
    base_specs = [
        pl.BlockSpec((p, bn, ef), lambda i: (0, i, 0)),
        pl.BlockSpec((bn, 3 * h), lambda i: (i, 0)),
        pl.BlockSpec((bn, h), lambda i: (i, 0)),
        pl.BlockSpec((ef, 3 * h), lambda i: (0, 0)),
        pl.BlockSpec((h, 3 * h), lambda i: (0, 0)),
        pl.BlockSpec((1, 3 * h), lambda i: (0, 0)),
    ]
    if not final:
        msg = w1st.shape[1]
        return pl.pallas_call(
            _gru_proj_body,
            grid=grid,
            in_specs=base_specs + [pl.BlockSpec((h, msg), lambda i: (0, 0)),
                                   pl.BlockSpec((h, msg), lambda i: (0, 0))],
            out_specs=[pl.BlockSpec((bn, h), lambda i: (i, 0)),
                       pl.BlockSpec((bn, msg), lambda i: (i, 0)),
                       pl.BlockSpec((bn, msg), lambda i: (i, 0))],
            out_shape=[jax.ShapeDtypeStruct((n, h), jnp.float32),
                       jax.ShapeDtypeStruct((n, msg), jnp.float32),
                       jax.ShapeDtypeStruct((n, msg), jnp.float32)],
        )(agg, gx0, states, wat, whht, bhh.reshape(1, -1), w1st, w1dt)
    nl = wft.shape[1]
    return pl.pallas_call(
        _gru_final_body,
        grid=grid,
        in_specs=base_specs + [pl.BlockSpec((h, nl), lambda i: (0, 0)),
                               pl.BlockSpec((1, nl), lambda i: (0, 0))],
        out_specs=pl.BlockSpec((bn, nl), lambda i: (i, 0)),
        out_shape=jax.ShapeDtypeStruct((n, nl), jnp.float32),
    )(agg, gx0, states, wat, whht, bhh.reshape(1, -1), wft, bf.reshape(1, -1))


def _sc_gather(ps, pd, srcc, dstc, e):
    h = ps.shape[1]
    n_chunks = e // _CH
    k_max = (n_chunks + _NW - 1) // _NW
    mesh = plsc.VectorSubcoreMesh(core_axis_name="c", subcore_axis_name="s")

    @functools.partial(
        pl.kernel, mesh=mesh,
        out_type=[jax.ShapeDtypeStruct((e, h), jnp.float32),
                  jax.ShapeDtypeStruct((e, h), jnp.float32)],
        scratch_types=[pltpu.VMEM((_CH,), jnp.int32),
                       pltpu.VMEM((_CH,), jnp.int32),
                       pltpu.VMEM((_CH, h), jnp.float32),
                       pltpu.VMEM((_CH, h), jnp.float32),
                       pltpu.SemaphoreType.DMA,
                       pltpu.SemaphoreType.DMA],
    )
    def g(ps_hbm, pd_hbm, srcc_hbm, dstc_hbm, os_hbm, od_hbm,
          idx_s, idx_d, rows_s, rows_d, sem_s, sem_d):
        w = lax.axis_index("s") * 2 + lax.axis_index("c")

        def body(k, carry):
            c = w + _NW * k

            @pl.when(c < n_chunks)
            def _():
                pltpu.sync_copy(srcc_hbm.at[pl.ds(c * _CH, _CH)], idx_s)
                pltpu.sync_copy(dstc_hbm.at[pl.ds(c * _CH, _CH)], idx_d)
                cp1 = pltpu.async_copy(ps_hbm.at[idx_s], rows_s, sem_s)
                cp2 = pltpu.async_copy(pd_hbm.at[idx_d], rows_d, sem_d)
                cp1.wait()
                cp2.wait()
                pltpu.sync_copy(rows_s, os_hbm.at[pl.ds(c * _CH, _CH)])
                pltpu.sync_copy(rows_d, od_hbm.at[pl.ds(c * _CH, _CH)])
            return carry

        lax.fori_loop(0, k_max, body, 0)

    return g(ps, pd, srcc, dstc)


def _sc_scatter(msgs, dstc, np_):
    e, ef = msgs.shape
    n_chunks = e // _CH
    k_max = (n_chunks + _NW - 1) // _NW
    rows_per_tile = np_ // _NT
    mesh = plsc.VectorSubcoreMesh(core_axis_name="c", subcore_axis_name="s")

    @functools.partial(
        pl.kernel, mesh=mesh,
        out_type=jax.ShapeDtypeStruct((2 * np_, ef), jnp.float32),
        scratch_types=[pltpu.VMEM((2, _CH), jnp.int32),
                       pltpu.VMEM((_CH, ef), jnp.float32),
                       pltpu.VMEM_SHARED((np_, ef), jnp.float32)],
    )
    def s(msgs_hbm, zeros_hbm, dstc_hbm, out_hbm, idx_d, msg_v, shared):
        cid = lax.axis_index("c")
        sid = lax.axis_index("s")
        w = sid * 2 + cid

        @pl.when(sid == 0)
        def _():
            pltpu.sync_copy(zeros_hbm, shared)

        plsc.subcore_barrier()

        def body(k, carry):
            c = w + _NW * k

            @pl.when(c < n_chunks)
            def _():
                pltpu.sync_copy(dstc_hbm.at[pl.ds(c * _CH, _CH)], idx_d.at[0])
                pltpu.sync_copy(msgs_hbm.at[pl.ds(c * _CH, _CH)], msg_v)
                pltpu.sync_copy(msg_v, shared.at[idx_d.at[0]], add=True)
            return carry

        lax.fori_loop(0, k_max, body, 0)
        plsc.subcore_barrier()

        @pl.when(sid == 0)
        def _():
            pltpu.sync_copy(shared, out_hbm.at[pl.ds(cid * np_, np_)])

    return s(msgs, jnp.zeros((np_, ef), jnp.float32),
             dstc).reshape(2, np_, ef)


def kernel(x_nodes, edge_index, edge_attr, node_mask, edge_mask,
           W1, b1, W2, b2, W3, b3, W4, b4, W_ih, b_ih, W_hh, b_hh, Wf, bf):
    n, h = x_nodes.shape
    e, ef = edge_attr.shape
    t_iters = 7
    src = edge_index[0].astype(jnp.int32)
    dst = edge_index[1].astype(jnp.int32)
    np_ = _rup(n, _BN)
    xp = jnp.pad(x_nodes, ((0, np_ - n), (0, 0)))

    w1st = W1[:, :h].T
    w1dt = W1[:, h:2 * h].T
    w1et = W1[:, 2 * h:].T
    wat = W_ih[:, :ef].T
    wxt = W_ih[:, ef:].T
    whht = W_hh.T
    wft = Wf.T

    gx0 = _tc_matmul_bias(xp, wxt, b_ih)

    srcc = src
    dstc = dst

    states = jnp.zeros((np_, h), jnp.float32)
    ps = pd = None
    out = None
    for t in range(t_iters):
        if t == 0:
            msgs = _tc_edge_mlp(None, None, edge_attr, w1et, b1, W2.T, b2,
                                W3.T, b3, W4.T, b4)
        else:
            pres, pred = _sc_gather(ps, pd, srcc, dstc, e)
            msgs = _tc_edge_mlp(pres, pred, edge_attr, w1et, b1, W2.T, b2,
                                W3.T, b3, W4.T, b4)
        agg = jnp.zeros((np_, ef), jnp.float32).at[dst].add(msgs)[None]
        if t < t_iters - 1:
            states, ps, pd = _tc_gru(agg, gx0, states, wat, whht, b_hh,
                                     final=False, w1st=w1st, w1dt=w1dt)
        else:
            out = _tc_gru(agg, gx0, states, wat, whht, b_hh,
                          final=True, wft=wft, bf=bf)
    return out[:n]


# speedup vs baseline: 2.2577x; 1.4029x over previous
import functools

import jax
import jax.numpy as jnp
from jax import lax
from jax.experimental import pallas as pl
from jax.experimental.pallas import tpu as pltpu
from jax.experimental.pallas import tpu_sc as plsc

_BN = 1024
_BE = 1600
_NW = 32
_NT = 16
_CH = 128


def _rup(x, m):
    return (x + m - 1) // m * m


def _matmul_bias_body(x_ref, wt_ref, b_ref, o_ref):
    o_ref[...] = (jnp.dot(x_ref[...], wt_ref[...],
                          preferred_element_type=jnp.float32) + b_ref[...])


def _tc_matmul_bias(x, wt, b):
    n, d1 = x.shape
    d2 = wt.shape[1]
    bn = min(_BN, n)
    return pl.pallas_call(
        _matmul_bias_body,
        grid=(n // bn,),
        in_specs=[pl.BlockSpec((bn, d1), lambda i: (i, 0)),
                  pl.BlockSpec((d1, d2), lambda i: (0, 0)),
                  pl.BlockSpec((1, d2), lambda i: (0, 0))],
        out_specs=pl.BlockSpec((bn, d2), lambda i: (i, 0)),
        out_shape=jax.ShapeDtypeStruct((n, d2), jnp.float32),
    )(x, wt, b.reshape(1, -1))


def _edge_mlp_body_pre(pres_ref, pred_ref, ea_ref, w1et_ref, b1_ref, w2t_ref,
                       b2_ref, w3t_ref, b3_ref, w4t_ref, b4_ref, o_ref):
    x = (jnp.dot(ea_ref[...], w1et_ref[...], preferred_element_type=jnp.float32)
         + b1_ref[...] + pres_ref[...] + pred_ref[...])
    hh = jax.nn.relu(x)
    hh = jax.nn.relu(jnp.dot(hh, w2t_ref[...], preferred_element_type=jnp.float32)
                     + b2_ref[...])
    hh = jax.nn.relu(jnp.dot(hh, w3t_ref[...], preferred_element_type=jnp.float32)
                     + b3_ref[...])
    o_ref[...] = (jnp.dot(hh, w4t_ref[...], preferred_element_type=jnp.float32)
                  + b4_ref[...])


def _edge_mlp_body_nopre(ea_ref, w1et_ref, b1_ref, w2t_ref, b2_ref,
                         w3t_ref, b3_ref, w4t_ref, b4_ref, o_ref):
    x = (jnp.dot(ea_ref[...], w1et_ref[...], preferred_element_type=jnp.float32)
         + b1_ref[...])
    hh = jax.nn.relu(x)
    hh = jax.nn.relu(jnp.dot(hh, w2t_ref[...], preferred_element_type=jnp.float32)
                     + b2_ref[...])
    hh = jax.nn.relu(jnp.dot(hh, w3t_ref[...], preferred_element_type=jnp.float32)
                     + b3_ref[...])
    o_ref[...] = (jnp.dot(hh, w4t_ref[...], preferred_element_type=jnp.float32)
                  + b4_ref[...])


def _tc_edge_mlp(pres, pred, ea, w1et, b1, w2t, b2, w3t, b3, w4t, b4):
    e, ef = ea.shape
    msg = w1et.shape[1]
    be = min(_BE, e)
    grid = (e // be,)
    wspecs = [
        pl.BlockSpec((ef, msg), lambda i: (0, 0)),
        pl.BlockSpec((1, msg), lambda i: (0, 0)),
        pl.BlockSpec((msg, msg), lambda i: (0, 0)),
        pl.BlockSpec((1, msg), lambda i: (0, 0)),
        pl.BlockSpec((msg, msg), lambda i: (0, 0)),
        pl.BlockSpec((1, msg), lambda i: (0, 0)),
        pl.BlockSpec((msg, ef), lambda i: (0, 0)),
        pl.BlockSpec((1, ef), lambda i: (0, 0)),
    ]
    wargs = (w1et, b1.reshape(1, -1), w2t, b2.reshape(1, -1),
             w3t, b3.reshape(1, -1), w4t, b4.reshape(1, -1))
    out_spec = pl.BlockSpec((be, ef), lambda i: (i, 0))
    out_shape = jax.ShapeDtypeStruct((e, ef), jnp.float32)
    if pres is None:
        return pl.pallas_call(
            _edge_mlp_body_nopre,
            grid=grid,
            in_specs=[pl.BlockSpec((be, ef), lambda i: (i, 0))] + wspecs,
            out_specs=out_spec, out_shape=out_shape,
        )(ea, *wargs)
    hdim = pres.shape[1]
    return pl.pallas_call(
        _edge_mlp_body_pre,
        grid=grid,
        in_specs=[pl.BlockSpec((be, hdim), lambda i: (i, 0)),
                  pl.BlockSpec((be, hdim), lambda i: (i, 0)),
                  pl.BlockSpec((be, ef), lambda i: (i, 0))] + wspecs,
        out_specs=out_spec, out_shape=out_shape,
    )(pres, pred, ea, *wargs)


def _gru_core(agg_ref, gx0_ref, st_ref, wat_ref, whht_ref, bhh_ref):
    h = st_ref.shape[1]
    agg = jnp.sum(agg_ref[...], axis=0)
    gx = (jnp.dot(agg, wat_ref[...], preferred_element_type=jnp.float32)
          + gx0_ref[...])
    gh = (jnp.dot(st_ref[...], whht_ref[...], preferred_element_type=jnp.float32)
          + bhh_ref[...])
    r = jax.nn.sigmoid(gx[:, :h] + gh[:, :h])
    z = jax.nn.sigmoid(gx[:, h:2 * h] + gh[:, h:2 * h])
    nn_ = jnp.tanh(gx[:, 2 * h:] + r * gh[:, 2 * h:])
    return (1.0 - z) * nn_ + z * st_ref[...]


def _gru_proj_body(agg_ref, gx0_ref, st_ref, wat_ref, whht_ref, bhh_ref,
                   w1st_ref, w1dt_ref, ns_ref, ps_ref, pd_ref):
    ns = _gru_core(agg_ref, gx0_ref, st_ref, wat_ref, whht_ref, bhh_ref)
    ns_ref[...] = ns
    ps_ref[...] = jnp.dot(ns, w1st_ref[...], preferred_element_type=jnp.float32)
    pd_ref[...] = jnp.dot(ns, w1dt_ref[...], preferred_element_type=jnp.float32)


def _gru_final_body(agg_ref, gx0_ref, st_ref, wat_ref, whht_ref, bhh_ref,
                    wft_ref, bf_ref, lg_ref):
    ns = _gru_core(agg_ref, gx0_ref, st_ref, wat_ref, whht_ref, bhh_ref)
    lg_ref[...] = (jnp.dot(ns, wft_ref[...], preferred_element_type=jnp.float32)
                   + bf_ref[...])


def _tc_gru(agg, gx0, states, wat, whht, bhh, final, w1st=None, w1dt=None,
            wft=None, bf=None):
    p, n, ef = agg.shape
    h = states.shape[1]
    bn = min(_BN, n)
    grid = (n // bn,)
    base_specs = [
        pl.BlockSpec((p, bn, ef), lambda i: (0, i, 0)),
        pl.BlockSpec((bn, 3 * h), lambda i: (i, 0)),
        pl.BlockSpec((bn, h), lambda i: (i, 0)),
        pl.BlockSpec((ef, 3 * h), lambda i: (0, 0)),
        pl.BlockSpec((h, 3 * h), lambda i: (0, 0)),
        pl.BlockSpec((1, 3 * h), lambda i: (0, 0)),
    ]
    if not final:
        msg = w1st.shape[1]
        return pl.pallas_call(
            _gru_proj_body,
            grid=grid,
            in_specs=base_specs + [pl.BlockSpec((h, msg), lambda i: (0, 0)),
                                   pl.BlockSpec((h, msg), lambda i: (0, 0))],
            out_specs=[pl.BlockSpec((bn, h), lambda i: (i, 0)),
                       pl.BlockSpec((bn, msg), lambda i: (i, 0)),
                       pl.BlockSpec((bn, msg), lambda i: (i, 0))],
            out_shape=[jax.ShapeDtypeStruct((n, h), jnp.float32),
                       jax.ShapeDtypeStruct((n, msg), jnp.float32),
                       jax.ShapeDtypeStruct((n, msg), jnp.float32)],
        )(agg, gx0, states, wat, whht, bhh.reshape(1, -1), w1st, w1dt)
    nl = wft.shape[1]
    return pl.pallas_call(
        _gru_final_body,
        grid=grid,
        in_specs=base_specs + [pl.BlockSpec((h, nl), lambda i: (0, 0)),
                               pl.BlockSpec((1, nl), lambda i: (0, 0))],
        out_specs=pl.BlockSpec((bn, nl), lambda i: (i, 0)),
        out_shape=jax.ShapeDtypeStruct((n, nl), jnp.float32),
    )(agg, gx0, states, wat, whht, bhh.reshape(1, -1), wft, bf.reshape(1, -1))


def _sc_gather(ps, pd, srcc, dstc, e):
    h = ps.shape[1]
    n_chunks = e // _CH
    k_max = (n_chunks + _NW - 1) // _NW
    mesh = plsc.VectorSubcoreMesh(core_axis_name="c", subcore_axis_name="s")

    @functools.partial(
        pl.kernel, mesh=mesh,
        out_type=[jax.ShapeDtypeStruct((e, h), jnp.float32),
                  jax.ShapeDtypeStruct((e, h), jnp.float32)],
        scratch_types=[pltpu.VMEM((_CH,), jnp.int32),
                       pltpu.VMEM((_CH,), jnp.int32),
                       pltpu.VMEM((_CH, h), jnp.float32),
                       pltpu.VMEM((_CH, h), jnp.float32),
                       pltpu.SemaphoreType.DMA,
                       pltpu.SemaphoreType.DMA],
    )
    def g(ps_hbm, pd_hbm, srcc_hbm, dstc_hbm, os_hbm, od_hbm,
          idx_s, idx_d, rows_s, rows_d, sem_s, sem_d):
        w = lax.axis_index("s") * 2 + lax.axis_index("c")

        def body(k, carry):
            c = w + _NW * k

            @pl.when(c < n_chunks)
            def _():
                pltpu.sync_copy(srcc_hbm.at[pl.ds(c * _CH, _CH)], idx_s)
                pltpu.sync_copy(dstc_hbm.at[pl.ds(c * _CH, _CH)], idx_d)
                cp1 = pltpu.async_copy(ps_hbm.at[idx_s], rows_s, sem_s)
                cp2 = pltpu.async_copy(pd_hbm.at[idx_d], rows_d, sem_d)
                cp1.wait()
                cp2.wait()
                pltpu.sync_copy(rows_s, os_hbm.at[pl.ds(c * _CH, _CH)])
                pltpu.sync_copy(rows_d, od_hbm.at[pl.ds(c * _CH, _CH)])
            return carry

        lax.fori_loop(0, k_max, body, 0)

    return g(ps, pd, srcc, dstc)


def _sc_scatter(msgs, dstc, np_):
    e, ef = msgs.shape
    n_chunks = e // _CH
    k_max = (n_chunks + _NW - 1) // _NW
    rows_per_tile = np_ // _NT
    mesh = plsc.VectorSubcoreMesh(core_axis_name="c", subcore_axis_name="s")

    @functools.partial(
        pl.kernel, mesh=mesh,
        out_type=jax.ShapeDtypeStruct((2 * np_, ef), jnp.float32),
        compiler_params=pltpu.CompilerParams(use_tc_tiling_on_sc=False),
        scratch_types=[pltpu.VMEM((_CH,), jnp.int32),
                       pltpu.VMEM((_CH, ef), jnp.float32),
                       pltpu.VMEM_SHARED((np_, ef), jnp.float32)],
    )
    def s(msgs_hbm, zeros_hbm, dstc_hbm, out_hbm, idx_d, msg_v, shared):
        cid = lax.axis_index("c")
        sid = lax.axis_index("s")
        w = sid * 2 + cid

        @pl.when(sid == 0)
        def _():
            pltpu.sync_copy(zeros_hbm, shared)

        plsc.subcore_barrier()

        def body(k, carry):
            c = w + _NW * k

            @pl.when(c < n_chunks)
            def _():
                pltpu.sync_copy(dstc_hbm.at[pl.ds(c * _CH, _CH)], idx_d)
                pltpu.sync_copy(msgs_hbm.at[pl.ds(c * _CH, _CH)], msg_v)
                pltpu.sync_copy(msg_v, shared.at[idx_d], add=True)
            return carry

        lax.fori_loop(0, k_max, body, 0)
        plsc.subcore_barrier()

        @pl.when(sid == 0)
        def _():
            pltpu.sync_copy(shared, out_hbm.at[pl.ds(cid * np_, np_)])

    return s(msgs, jnp.zeros((np_, ef), jnp.float32),
             dstc).reshape(2, np_, ef)


def kernel(x_nodes, edge_index, edge_attr, node_mask, edge_mask,
           W1, b1, W2, b2, W3, b3, W4, b4, W_ih, b_ih, W_hh, b_hh, Wf, bf):
    n, h = x_nodes.shape
    e, ef = edge_attr.shape
    t_iters = 7
    src = edge_index[0].astype(jnp.int32)
    dst = edge_index[1].astype(jnp.int32)
    np_ = _rup(n, _BN)
    xp = jnp.pad(x_nodes, ((0, np_ - n), (0, 0)))

    w1st = W1[:, :h].T
    w1dt = W1[:, h:2 * h].T
    w1et = W1[:, 2 * h:].T
    wat = W_ih[:, :ef].T
    wxt = W_ih[:, ef:].T
    whht = W_hh.T
    wft = Wf.T

    gx0 = _tc_matmul_bias(xp, wxt, b_ih)

    srcc = src
    dstc = dst

    states = jnp.zeros((np_, h), jnp.float32)
    ps = pd = None
    out = None
    for t in range(t_iters):
        if t == 0:
            msgs = _tc_edge_mlp(None, None, edge_attr, w1et, b1, W2.T, b2,
                                W3.T, b3, W4.T, b4)
        else:
            pres, pred = _sc_gather(ps, pd, srcc, dstc, e)
            msgs = _tc_edge_mlp(pres, pred, edge_attr, w1et, b1, W2.T, b2,
                                W3.T, b3, W4.T, b4)
        agg = _sc_scatter(msgs, dstc, np_)
        if t < t_iters - 1:
            states, ps, pd = _tc_gru(agg, gx0, states, wat, whht, b_hh,
                                     final=False, w1st=w1st, w1dt=w1dt)
        else:
            out = _tc_gru(agg, gx0, states, wat, whht, b_hh,
                          final=True, wft=wft, bf=bf)
    return out[:n]
